# Initial kernel scaffold; baseline (speedup 1.0000x reference)
import jax
import jax.numpy as jnp
from jax.experimental import pallas as pl


def kernel(embeddings, labels):
    raise NotImplementedError("write your pallas kernel here")


# speedup vs baseline: 136.8077x; 136.8077x over previous
import functools

import jax
import jax.numpy as jnp
from jax import lax
from jax.experimental import pallas as pl
from jax.experimental.pallas import tpu as pltpu

_MARGIN = 1.0
_K = 256
_EPS = 1e-12


def _body(emb_ref, lab_col_ref, lab_row_ref, out_ref, pos_ref, neg_ref):
    emb = emb_ref[...]
    B = emb.shape[0]

    g = lax.dot_general(emb, emb, (((1,), (1,)), ((), ())),
                        preferred_element_type=jnp.float32)

    row = lax.broadcasted_iota(jnp.int32, (B, B), 0)
    col = lax.broadcasted_iota(jnp.int32, (B, B), 1)
    diag = row == col
    upper = row < col

    gd = jnp.where(diag, g, 0.0)
    norms_col = jnp.sum(gd, axis=1, keepdims=True)
    norms_row = jnp.sum(gd, axis=0, keepdims=True)

    sq = jnp.maximum(norms_col + norms_row - 2.0 * g, 0.0)

    labmatch = lab_col_ref[...] == lab_row_ref[...]
    posmask = labmatch & upper
    negmask = (~labmatch) & upper

    n_pos = jnp.sum(posmask.astype(jnp.float32))
    n_neg = jnp.sum(negmask.astype(jnp.float32))

    pos_ref[...] = jnp.where(posmask, sq, -1.0)
    neg_ref[...] = jnp.where(negmask, sq, jnp.inf)

    kf = jnp.float32(_K)

    def search_step(_, carry):
        lo_p, hi_p, lo_n, hi_n = carry
        mid_p = lo_p + (hi_p - lo_p) // 2
        mid_n = lo_n + (hi_n - lo_n) // 2
        t_p = lax.bitcast_convert_type(mid_p, jnp.float32)
        t_n = lax.bitcast_convert_type(mid_n, jnp.float32)
        cnt_p = jnp.sum((pos_ref[...] >= t_p).astype(jnp.float32))
        cnt_n = jnp.sum((neg_ref[...] <= t_n).astype(jnp.float32))
        lo_p = jnp.where(cnt_p >= kf, mid_p, lo_p)
        hi_p = jnp.where(cnt_p >= kf, hi_p, mid_p)
        hi_n = jnp.where(cnt_n >= kf, mid_n, hi_n)
        lo_n = jnp.where(cnt_n >= kf, lo_n, mid_n)
        return lo_p, hi_p, lo_n, hi_n

    init = (jnp.int32(0), jnp.int32(0x7f800000),
            jnp.int32(-1), jnp.int32(0x7f7fffff))
    lo_p, hi_p, lo_n, hi_n = lax.fori_loop(0, 31, search_step, init)

    t_pos = lax.bitcast_convert_type(lo_p, jnp.float32)
    t_neg = lax.bitcast_convert_type(hi_n, jnp.float32)

    posvals = pos_ref[...]
    negvals = neg_ref[...]

    def pos_term(v):
        d = jnp.sqrt(v + _EPS)
        return d * d

    def neg_term(v):
        r = jnp.maximum(_MARGIN - jnp.sqrt(v + _EPS), 0.0)
        return r * r

    pgt = posvals > t_pos
    cnt_pgt = jnp.sum(pgt.astype(jnp.float32))
    sum_pgt = jnp.sum(jnp.where(pgt, pos_term(posvals), 0.0))
    pos_topk = sum_pgt + (kf - cnt_pgt) * pos_term(t_pos)
    pos_all = jnp.sum(jnp.where(posvals >= 0.0, pos_term(posvals), 0.0))
    pos_sum = jnp.where(n_pos > kf, pos_topk, pos_all)

    nlt = negvals < t_neg
    cnt_nlt = jnp.sum(nlt.astype(jnp.float32))
    sum_nlt = jnp.sum(jnp.where(nlt, neg_term(negvals), 0.0))
    neg_topk = sum_nlt + (kf - cnt_nlt) * neg_term(t_neg)
    neg_all = jnp.sum(jnp.where(negvals < jnp.inf, neg_term(negvals), 0.0))
    neg_sum = jnp.where(n_neg > kf, neg_topk, neg_all)

    count = jnp.minimum(n_pos, kf) + jnp.minimum(n_neg, kf)
    out_ref[0, 0] = (pos_sum + neg_sum) / count


@jax.jit
def kernel(embeddings, labels):
    B = embeddings.shape[0]
    labels = labels.astype(jnp.int32)
    lab_col = labels.reshape(B, 1)
    lab_row = labels.reshape(1, B)
    out = pl.pallas_call(
        _body,
        out_shape=jax.ShapeDtypeStruct((1, 1), jnp.float32),
        out_specs=pl.BlockSpec(memory_space=pltpu.SMEM),
        scratch_shapes=[
            pltpu.VMEM((B, B), jnp.float32),
            pltpu.VMEM((B, B), jnp.float32),
        ],
    )(embeddings, lab_col, lab_row)
    return out.reshape(())


# speedup vs baseline: 224.2608x; 1.6392x over previous
import jax
import jax.numpy as jnp
from jax import lax
from jax.experimental import pallas as pl
from jax.experimental.pallas import tpu as pltpu

_MARGIN = 1.0
_K = 256
_EPS = 1e-12
_INT_MIN = -2147483648
_SENT = 0x7f800000
_BLK = 128


def _body(emb_ref, lab_col_ref, lab_row_ref, out_ref, full_ref, enc_ref):
    emb = emb_ref[...]
    B = emb.shape[0]
    nblk = B // _BLK
    n_sent = nblk * (_BLK * (_BLK + 1) // 2)

    g = lax.dot_general(emb, emb, (((1,), (1,)), ((), ())),
                        preferred_element_type=jnp.float32)

    row = lax.broadcasted_iota(jnp.int32, (B, B), 0)
    col = lax.broadcasted_iota(jnp.int32, (B, B), 1)
    gd = jnp.where(row == col, g, 0.0)
    norms_col = jnp.sum(gd, axis=1, keepdims=True)
    norms_row = jnp.sum(gd, axis=0, keepdims=True)

    sq = jnp.maximum(norms_col + norms_row - 2.0 * g, 0.0)
    labmatch = lab_col_ref[...] == lab_row_ref[...]

    nmatch = jnp.sum(labmatch.astype(jnp.float32))
    n_pos = (nmatch - B) * 0.5
    n_neg = (B * B - nmatch) * 0.5

    bits = jnp.maximum(lax.bitcast_convert_type(sq, jnp.int32), 0)
    enc = jnp.where(labmatch, bits, bits + jnp.int32(_INT_MIN))
    full_ref[...] = jnp.where(row < col, enc, jnp.int32(_SENT))

    idx = 0
    for bi in range(nblk):
        for bj in range(bi, nblk):
            enc_ref[idx * _BLK:(idx + 1) * _BLK, :] = (
                full_ref[bi * _BLK:(bi + 1) * _BLK,
                         bj * _BLK:(bj + 1) * _BLK])
            idx += 1

    kf = jnp.float32(_K)
    sentf = jnp.float32(n_sent)

    def search_step(_, carry):
        lo_p, hi_p, lo_n, hi_n = carry
        mid_p = lo_p + (hi_p - lo_p) // 2
        mid_n = lo_n + (hi_n - lo_n) // 2
        e = enc_ref[...]
        cnt_p = jnp.sum(jnp.where(e >= mid_p, 1.0, 0.0)) - sentf
        cnt_n = jnp.sum(jnp.where(e <= mid_n, 1.0, 0.0))
        lo_p = jnp.where(cnt_p >= kf, mid_p, lo_p)
        hi_p = jnp.where(cnt_p >= kf, hi_p, mid_p)
        hi_n = jnp.where(cnt_n >= kf, mid_n, hi_n)
        lo_n = jnp.where(cnt_n >= kf, lo_n, mid_n)
        return lo_p, hi_p, lo_n, hi_n

    init = (jnp.int32(0), jnp.int32(_SENT),
            jnp.int32(_INT_MIN), jnp.int32(_INT_MIN + _SENT))
    lo_p, hi_p, lo_n, hi_n = lax.fori_loop(0, 31, search_step, init)

    t_pos = lax.bitcast_convert_type(lo_p, jnp.float32)
    sbits = hi_n - jnp.int32(_INT_MIN)
    t_neg = lax.bitcast_convert_type(sbits, jnp.float32)

    e = enc_ref[...]
    v = lax.bitcast_convert_type(
        jnp.where(e >= 0, e, e - jnp.int32(_INT_MIN)), jnp.float32)

    pgt = (e > lo_p) & (e < jnp.int32(_SENT))
    cnt_pgt = jnp.sum(jnp.where(pgt, 1.0, 0.0))
    sum_pgt = jnp.sum(jnp.where(pgt, v, 0.0))
    pos_sum = sum_pgt + (kf - cnt_pgt) * t_pos + kf * _EPS

    isneg = e < 0
    f_active = isneg & (v < _MARGIN * _MARGIN)
    d = jnp.sqrt(v + _EPS)
    r = jnp.maximum(_MARGIN - d, 0.0)
    fv = jnp.where(f_active, r * r, 0.0)
    nlt = e < hi_n
    cnt_nlt = jnp.sum(jnp.where(nlt, 1.0, 0.0))
    sum_nlt = jnp.sum(jnp.where(nlt, fv, 0.0))
    dt = jnp.sqrt(t_neg + _EPS)
    rt = jnp.maximum(_MARGIN - dt, 0.0)
    neg_topk = sum_nlt + (kf - cnt_nlt) * rt * rt
    neg_all = jnp.sum(fv)
    neg_sum = jnp.where(n_neg > kf, neg_topk, neg_all)

    count = jnp.minimum(n_pos, kf) + jnp.minimum(n_neg, kf)
    out_ref[0, 0] = (pos_sum + neg_sum) / count


@jax.jit
def kernel(embeddings, labels):
    B = embeddings.shape[0]
    nblk = B // _BLK
    npairs_blk = nblk * (nblk + 1) // 2
    labels = labels.astype(jnp.int32)
    lab_col = labels.reshape(B, 1)
    lab_row = labels.reshape(1, B)
    out = pl.pallas_call(
        _body,
        out_shape=jax.ShapeDtypeStruct((1, 1), jnp.float32),
        out_specs=pl.BlockSpec(memory_space=pltpu.SMEM),
        scratch_shapes=[
            pltpu.VMEM((B, B), jnp.int32),
            pltpu.VMEM((npairs_blk * _BLK, _BLK), jnp.int32),
        ],
    )(embeddings, lab_col, lab_row)
    return out.reshape(())
